# Initial kernel scaffold; baseline (speedup 1.0000x reference)
import jax
import jax.numpy as jnp
from jax.experimental import pallas as pl


def kernel(X, grid_part, grid_part_norm, int_map, allcombo_idx, idx_map):
    raise NotImplementedError("write your pallas kernel here")


# speedup vs baseline: 1.8720x; 1.8720x over previous
import functools

import jax
import jax.numpy as jnp
from jax import lax
from jax.experimental import pallas as pl
from jax.experimental.pallas import tpu as pltpu
from jax.experimental.pallas import tpu_sc as plsc

_N = 262144
_CODESZ = 8
_MP = 3072
_B = 512
_NW = 32
_CH = 128


def _tc_body(x_ref, g2tn_ref, w_ref, imap_ref, intmap_ref, vals_ref, flat_ref, *, m_real):
    x = x_ref[...]
    neg = x < 0.0
    negi = neg.astype(jnp.int32)
    odd = (jnp.sum(negi, axis=1, keepdims=True) & 1) == 1
    col = lax.broadcasted_iota(jnp.int32, x.shape, 1)
    flip0 = jnp.where((col == 0) & odd, -1.0, 1.0)
    xp = jnp.abs(x) * flip0
    mask = (1.0 - 2.0 * neg.astype(jnp.float32)) * flip0

    xp_aug = jnp.concatenate([xp, jnp.ones_like(x[:, :1])], axis=1)
    scores = jnp.dot(xp_aug, g2tn_ref[...],
                     preferred_element_type=jnp.float32)
    smax = jnp.max(scores, axis=1, keepdims=True)
    lane = lax.broadcasted_iota(jnp.int32, scores.shape, 1)
    qidx = jnp.min(jnp.where(scores == smax, lane, _MP), axis=1)[:, None]
    onehot = (lane == qidx).astype(jnp.float32)

    sel = jnp.dot(onehot, w_ref[...], preferred_element_type=jnp.float32)
    vals_ref[...] = sel[:, :_CODESZ] * mask

    signbit = (mask < 0.0).astype(jnp.int32)
    sign_int = jnp.sum(signbit * intmap_ref[...], axis=1, keepdims=True)
    io256 = lax.broadcasted_iota(jnp.int32, (x.shape[0], 256), 1)
    oh256 = (io256 == sign_int).astype(jnp.float32)
    rowf = jnp.dot(oh256, imap_ref[...], preferred_element_type=jnp.float32)
    flat_ref[...] = rowf.astype(jnp.int32) * m_real + qidx


def _tc_call(X, g2tn, w, imap, intmap, m_real, interpret=False):
    n = X.shape[0]
    grid = n // _B
    return pl.pallas_call(
        functools.partial(_tc_body, m_real=m_real),
        grid=(grid,),
        in_specs=[
            pl.BlockSpec((_B, _CODESZ), lambda i: (i, 0)),
            pl.BlockSpec(g2tn.shape, lambda i: (0, 0)),
            pl.BlockSpec(w.shape, lambda i: (0, 0)),
            pl.BlockSpec(imap.shape, lambda i: (0, 0)),
            pl.BlockSpec(intmap.shape, lambda i: (0, 0)),
        ],
        out_specs=[
            pl.BlockSpec((_B, _CODESZ), lambda i: (i, 0)),
            pl.BlockSpec((_B, 1), lambda i: (i, 0)),
        ],
        out_shape=[
            jax.ShapeDtypeStruct((n, _CODESZ), jnp.float32),
            jax.ShapeDtypeStruct((n, 1), jnp.int32),
        ],
        compiler_params=pltpu.CompilerParams(
            dimension_semantics=("arbitrary",),
        ),
        interpret=interpret,
    )(X, g2tn, w, imap, intmap)


def _sc_gather(flat_idx2d, table):
    nrows, ch = flat_idx2d.shape
    r_per_w = nrows // _NW
    mesh = plsc.VectorSubcoreMesh(core_axis_name="c", subcore_axis_name="s")

    @functools.partial(
        pl.kernel,
        mesh=mesh,
        out_type=jax.ShapeDtypeStruct((nrows, ch), jnp.int32),
        scratch_types=[
            pltpu.VMEM((r_per_w, _CH), jnp.int32),
            pltpu.VMEM((r_per_w, _CH), jnp.int32),
            pltpu.SemaphoreType.DMA,
        ],
    )
    def gather_k(idx_hbm, table_hbm, out_hbm, idx_v, out_v, sem):
        wid = lax.axis_index("s") * 2 + lax.axis_index("c")
        base = wid * r_per_w
        pltpu.sync_copy(idx_hbm.at[pl.ds(base, r_per_w)], idx_v)

        def body(i, carry):
            cp = pltpu.async_copy(
                table_hbm.at[idx_v.at[i]],
                out_v.at[i],
                sem,
            )
            cp.wait()
            return carry

        lax.fori_loop(0, r_per_w, body, 0)
        pltpu.sync_copy(out_v, out_hbm.at[pl.ds(base, r_per_w)])

    return gather_k(flat_idx2d, table)


def _prep(grid_part, grid_part_norm, int_map, idx_map):
    m = grid_part.shape[0]
    g2t = jnp.zeros((_CODESZ + 1, _MP), jnp.float32)
    g2t = g2t.at[:_CODESZ, :m].set(2.0 * grid_part.T)
    g2tn = g2t.at[_CODESZ, :].set(
        jnp.full((_MP,), -1e30, jnp.float32).at[:m].set(-grid_part_norm))
    w = jnp.zeros((_MP, 16), jnp.float32)
    w = w.at[:m, :_CODESZ].set(grid_part)
    imap = idx_map.astype(jnp.float32).reshape(256, 1)
    intmap = int_map.astype(jnp.int32).reshape(1, _CODESZ)
    return g2tn, w, imap, intmap


def kernel(X, grid_part, grid_part_norm, int_map, allcombo_idx, idx_map):
    m = grid_part.shape[0]
    g2tn, w, imap, intmap = _prep(grid_part, grid_part_norm, int_map, idx_map)
    vals, flat = _tc_call(X, g2tn, w, imap, intmap, m)
    table = allcombo_idx.astype(jnp.int32).reshape(-1)
    real_idx = _sc_gather(flat.reshape(-1, _CH), table)
    return vals, real_idx.reshape(-1).astype(jnp.int16)
